# baseline (speedup 1.0000x reference)
import jax
import jax.numpy as jnp
from jax.experimental import pallas as pl
from jax.experimental.pallas import tpu as pltpu

NB = 2
TOP_K = 2


def _moe_body(x_ref, wg_ref, w1a_ref, w1b_ref, w3a_ref, w3b_ref, w2a_ref,
              w2b_ref, out_ref, route_ref, a_ref):
    e = pl.program_id(0)
    nb = pl.program_id(1)
    n_e = route_ref.shape[1]
    f = a_ref.shape[2]
    f_blk = f // NB
    f_half = f_blk // 2
    d_blk = out_ref.shape[1] // NB
    d_half = d_blk // 2

    @pl.when(jnp.logical_and(e == 0, nb == 0))
    def _init():
        xv = x_ref[...]
        logits = jax.lax.dot_general(
            xv, wg_ref[...], (((1,), (0,)), ((), ())),
            preferred_element_type=jnp.float32)
        mx = jnp.max(logits, axis=-1, keepdims=True)
        pr = jnp.exp(logits - mx)
        pr = pr / jnp.sum(pr, axis=-1, keepdims=True)
        ecols = jax.lax.broadcasted_iota(jnp.int32, pr.shape, 1)
        m1 = jnp.max(pr, axis=-1, keepdims=True)
        i1 = jnp.argmax(pr, axis=-1)[:, None]
        masked = jnp.where(ecols == i1, -jnp.inf, pr)
        m2 = jnp.max(masked, axis=-1, keepdims=True)
        i2 = jnp.argmax(masked, axis=-1)[:, None]
        s = m1 + m2
        route_ref[...] = jnp.where(
            ecols == i1, m1 / s, jnp.where(ecols == i2, m2 / s, 0.0))
        out_ref[...] = jnp.zeros_like(out_ref)

    @pl.when(e < n_e)
    def _up_proj():
        xv = x_ref[...]
        ecols = jax.lax.broadcasted_iota(jnp.int32, route_ref.shape, 1)
        rw = jnp.sum(jnp.where(ecols == e, route_ref[...], 0.0), axis=1,
                     keepdims=True)
        for half, wref1, wref3 in ((0, w1a_ref, w3a_ref),
                                   (1, w1b_ref, w3b_ref)):
            g = jax.lax.dot_general(
                xv, wref1[0], (((1,), (1,)), ((), ())),
                preferred_element_type=jnp.float32)
            u = jax.lax.dot_general(
                xv, wref3[0], (((1,), (1,)), ((), ())),
                preferred_element_type=jnp.float32)
            a_ref[e % 2, :, pl.ds(nb * f_blk + half * f_half, f_half)] = (
                (g * jax.lax.logistic(g)) * u * rw)

    @pl.when(e > 0)
    def _down_proj():
        a_prev = a_ref[(e - 1) % 2]
        for half, wref2 in ((0, w2a_ref), (1, w2b_ref)):
            y = jax.lax.dot_general(
                a_prev, wref2[0], (((1,), (1,)), ((), ())),
                preferred_element_type=jnp.float32)
            out_ref[:, pl.ds(nb * d_blk + half * d_half, d_half)] += y


@jax.jit
def kernel(x, Wg, w1, w3, w2):
    m, d = x.shape
    e_num = Wg.shape[1]
    f = w1.shape[1]
    f_half = f // NB // 2
    d_half = d // NB // 2

    def up_map(half):
        def _map(e, nb):
            return (jnp.minimum(e, e_num - 1),
                    jnp.where(e < e_num, 2 * nb + half, 2 * (NB - 1) + half), 0)
        return _map

    def down_map(half):
        def _map(e, nb):
            return (jnp.maximum(e - 1, 0),
                    jnp.where(e == 0, half, 2 * nb + half), 0)
        return _map

    return pl.pallas_call(
        _moe_body,
        grid=(e_num + 1, NB),
        in_specs=[
            pl.BlockSpec((m, d), lambda e, nb: (0, 0)),
            pl.BlockSpec((d, e_num), lambda e, nb: (0, 0)),
            pl.BlockSpec((1, f_half, d), up_map(0)),
            pl.BlockSpec((1, f_half, d), up_map(1)),
            pl.BlockSpec((1, f_half, d), up_map(0)),
            pl.BlockSpec((1, f_half, d), up_map(1)),
            pl.BlockSpec((1, d_half, f), down_map(0)),
            pl.BlockSpec((1, d_half, f), down_map(1)),
        ],
        out_specs=pl.BlockSpec((m, d), lambda e, nb: (0, 0)),
        out_shape=jax.ShapeDtypeStruct((m, d), x.dtype),
        scratch_shapes=[
            pltpu.VMEM((m, e_num), jnp.float32),
            pltpu.VMEM((2, m, f), jnp.float32),
        ],
    )(x, Wg, w1, w1, w3, w3, w2, w2)
